# Initial kernel scaffold; baseline (speedup 1.0000x reference)
import jax
import jax.numpy as jnp
from jax.experimental import pallas as pl


def kernel(hidden_states, attention_mask, W, b):
    raise NotImplementedError("write your pallas kernel here")


# speedup vs baseline: 2.5042x; 2.5042x over previous
import jax
import jax.numpy as jnp
from jax.experimental import pallas as pl
from jax.experimental.pallas import tpu as pltpu

_THRESHOLD = 0.95
_MV_BLK = 1024


def _mv_kernel(h_ref, w_ref, o_ref):
    o_ref[...] = jax.lax.dot_general(
        h_ref[...], w_ref[...], (((1,), (0,)), ((), ())),
        preferred_element_type=jnp.float32)


def _shift_right(x, j):
    b, n = x.shape
    return jnp.concatenate(
        [jnp.zeros((b, j), x.dtype), x[:, :n - j]], axis=1)


def _shift_left(x, j):
    b, n = x.shape
    return jnp.concatenate(
        [x[:, j:], jnp.zeros((b, j), x.dtype)], axis=1)


def _row_total(gated, nl=256):
    b, n = gated.shape
    chunks = [gated[:, i * nl:(i + 1) * nl] for i in range(n // nl)]
    while len(chunks) > 1:
        chunks = [chunks[2 * i] + chunks[2 * i + 1]
                  for i in range(len(chunks) // 2)]
    t = chunks[0]
    w = nl
    while w > 1:
        w //= 2
        t = t[:, :w] + t[:, w:2 * w]
    return t


def _bitonic_desc(x, col):
    b, n = x.shape
    k = 2
    while k <= n:
        dir_desc = (col & k) == 0
        j = k // 2
        while j >= 1:
            is_lower = (col & j) == 0
            partner = jnp.where(is_lower, _shift_left(x, j),
                                _shift_right(x, j))
            mx = jnp.maximum(x, partner)
            mn = jnp.minimum(x, partner)
            x = jnp.where(dir_desc == is_lower, mx, mn)
            j //= 2
        k *= 2
    return x


def _mask_kernel(logits_ref, am_ref, gates_ref, keep_ref, xt_ref, cumt_ref):
    b, n = logits_ref.shape
    nblk = n // 128
    ncols = b * nblk

    gates = jax.nn.sigmoid(logits_ref[...])
    gates_ref[...] = gates
    act = am_ref[...] != 0
    gated = jnp.where(act, gates, jnp.float32(0.0))

    total = jnp.maximum(_row_total(gated), jnp.float32(1e-12))
    shares = jnp.where(act, gated / total, jnp.float32(0.0))

    col = jax.lax.broadcasted_iota(jnp.int32, (b, n), 1)
    srt = _bitonic_desc(shares, col)

    xt_ref[...] = srt.reshape(ncols, 128).T

    def body(i, acc):
        acc = acc + xt_ref[pl.ds(i, 1), :]
        cumt_ref[pl.ds(i, 1), :] = acc
        return acc

    tot = jax.lax.fori_loop(
        0, 128, body, jnp.zeros((1, ncols), jnp.float32))

    lane = jax.lax.broadcasted_iota(jnp.int32, (1, ncols), 1)
    blk = lane & (nblk - 1)
    s = tot
    for step in range(1, nblk):
        s = jnp.where(blk == step, s + _shift_right(s, 1), s)
    ex = jnp.where(blk == 0, jnp.float32(0.0), _shift_right(s, 1))

    cum = (cumt_ref[...] + ex).T.reshape(b, n)

    k0 = jnp.sum((cum <= jnp.float32(_THRESHOLD)).astype(jnp.float32),
                 axis=1, keepdims=True)
    kk = jnp.maximum(k0, jnp.float32(1.0))

    sel = col == (kk.astype(jnp.int32) - 1)
    sstar = jnp.max(jnp.where(sel, srt, jnp.float32(-1.0)),
                    axis=1, keepdims=True)
    n_greater = jnp.sum((srt > sstar).astype(jnp.float32),
                        axis=1, keepdims=True)
    m = kk - n_greater

    eq = shares == sstar
    p = eq.astype(jnp.float32)
    d = 1
    while d < n:
        p = p + _shift_right(p, d)
        d *= 2

    keep = act & ((shares > sstar) | (eq & (p <= m)))
    keep_ref[...] = keep.astype(jnp.int32)


def kernel(hidden_states, attention_mask, W, b):
    bb, s, d = hidden_states.shape
    h2d = hidden_states.reshape(bb * s, d)

    mv = pl.pallas_call(
        _mv_kernel,
        grid=(bb * s // _MV_BLK,),
        in_specs=[pl.BlockSpec((_MV_BLK, d), lambda i: (i, 0)),
                  pl.BlockSpec((d, 1), lambda i: (0, 0))],
        out_specs=pl.BlockSpec((_MV_BLK, 1), lambda i: (i, 0)),
        out_shape=jax.ShapeDtypeStruct((bb * s, 1), jnp.float32),
    )(h2d, W.reshape(d, 1))

    logits = mv.reshape(bb, s) + b

    gates, keep = pl.pallas_call(
        _mask_kernel,
        out_shape=[jax.ShapeDtypeStruct((bb, s), jnp.float32),
                   jax.ShapeDtypeStruct((bb, s), jnp.int32)],
        scratch_shapes=[pltpu.VMEM((128, bb * s // 128), jnp.float32),
                        pltpu.VMEM((128, bb * s // 128), jnp.float32)],
    )(logits, attention_mask)

    return (logits, gates, keep.astype(jnp.bool_))
